# baseline (speedup 1.0000x reference)
import functools

import jax
import jax.numpy as jnp
from jax import lax
from jax.experimental import pallas as pl
from jax.experimental.pallas import tpu as pltpu
from jax.experimental.pallas import tpu_sc as plsc

_HIDDEN = 896
_CODEBOOK = 2000
_CODEBOOK_PAD = 2048
_MAX_PROMPT_LEN = 128
_BATCH = 16384
_TILE = 2048

_NC = 2
_NS = 16
_NW = _NC * _NS
_B_PER_W = _BATCH // _NW
_CH = 64
_NCH = _B_PER_W // _CH


def _tc_body(z_ref, emb_ref, mb0_ref, mb1_ref, w0_ref, w1_ref,
             idx_ref, l_ref, esq_ref):
    @pl.when(pl.program_id(0) == 0)
    def _():
        emb0 = emb_ref[...]
        esq = jnp.sum(emb0 * emb0, axis=1)[None, :]
        col = jax.lax.broadcasted_iota(jnp.int32, (1, _CODEBOOK_PAD), 1)
        esq_ref[...] = jnp.where(col < _CODEBOOK, esq, jnp.inf)

    z = z_ref[...]
    emb = emb_ref[...]
    mm = jax.lax.dot_general(z, emb, (((1,), (1,)), ((), ())),
                             preferred_element_type=jnp.float32)
    zz = jnp.sum(z * z, axis=1, keepdims=True)
    dist = zz - 2.0 * mm + esq_ref[...]
    dmin = jnp.min(dist, axis=1, keepdims=True)
    col = jax.lax.broadcasted_iota(jnp.int32, (_TILE, _CODEBOOK_PAD), 1)
    idx_ref[...] = jnp.min(jnp.where(dist == dmin, col, _CODEBOOK_PAD),
                           axis=1, keepdims=True)
    l_ref[:, 0, :] = jax.lax.dot_general(
        z, w0_ref[...], (((1,), (1,)), ((), ())),
        preferred_element_type=jnp.float32) + mb0_ref[...]
    l_ref[:, 1, :] = jax.lax.dot_general(
        z, w1_ref[...], (((1,), (1,)), ((), ())),
        preferred_element_type=jnp.float32) + mb1_ref[...]


def _sc_gather_body(idx_hbm, emb_hbm, out_hbm, idx_v, rows_v, gs0, gs1, ss0, ss1):
    wid = lax.axis_index("s") * _NC + lax.axis_index("c")
    base = wid * _B_PER_W
    pltpu.sync_copy(idx_hbm.at[wid], idx_v)
    gsems = (gs0, gs1)
    ssems = (ss0, ss1)
    gh = [None] * _NCH
    sh = [None] * _NCH
    gh[0] = pltpu.async_copy(emb_hbm.at[idx_v.at[0]], rows_v.at[0], gsems[0])
    for j in range(_NCH):
        b = j % 2
        if j + 1 < _NCH:
            if j - 1 >= 0:
                sh[j - 1].wait()
            gh[j + 1] = pltpu.async_copy(
                emb_hbm.at[idx_v.at[j + 1]], rows_v.at[(j + 1) % 2],
                gsems[(j + 1) % 2])
        gh[j].wait()
        sh[j] = pltpu.async_copy(
            rows_v.at[b], out_hbm.at[pl.ds(base + j * _CH, _CH)], ssems[b])
    sh[_NCH - 2].wait()
    sh[_NCH - 1].wait()


@jax.jit
def kernel(s2_premise_state, emb, W0, b0, W1, b1, prompt_len):
    z = s2_premise_state
    mask = jnp.where(jnp.arange(_MAX_PROMPT_LEN) < prompt_len,
                     jnp.float32(0.0), jnp.float32(-jnp.inf))
    mb0 = (b0 + mask)[None, :]
    mb1 = (b1 + mask)[None, :]
    emb_pad = jnp.pad(emb, ((0, _CODEBOOK_PAD - _CODEBOOK), (0, 0)))

    grid = _BATCH // _TILE
    idx, ptr_logits = pl.pallas_call(
        _tc_body,
        grid=(grid,),
        in_specs=[
            pl.BlockSpec((_TILE, _HIDDEN), lambda i: (i, 0)),
            pl.BlockSpec((_CODEBOOK_PAD, _HIDDEN), lambda i: (0, 0)),
            pl.BlockSpec((1, _MAX_PROMPT_LEN), lambda i: (0, 0)),
            pl.BlockSpec((1, _MAX_PROMPT_LEN), lambda i: (0, 0)),
            pl.BlockSpec((_MAX_PROMPT_LEN, _HIDDEN), lambda i: (0, 0)),
            pl.BlockSpec((_MAX_PROMPT_LEN, _HIDDEN), lambda i: (0, 0)),
        ],
        out_specs=[
            pl.BlockSpec((_TILE, 1), lambda i: (i, 0)),
            pl.BlockSpec((_TILE, 2, _MAX_PROMPT_LEN), lambda i: (i, 0, 0)),
        ],
        out_shape=[
            jax.ShapeDtypeStruct((_BATCH, 1), jnp.int32),
            jax.ShapeDtypeStruct((_BATCH, 2, _MAX_PROMPT_LEN), jnp.float32),
        ],
        scratch_shapes=[pltpu.VMEM((1, _CODEBOOK_PAD), jnp.float32)],
    )(z, emb_pad, mb0, mb1, W0, W1)

    gather = pl.kernel(
        _sc_gather_body,
        out_type=jax.ShapeDtypeStruct((_BATCH, _HIDDEN), jnp.float32),
        mesh=plsc.VectorSubcoreMesh(core_axis_name="c", subcore_axis_name="s"),
        scratch_types=[
            pltpu.VMEM((_NCH, _CH), jnp.int32),
            pltpu.VMEM((2, _CH, _HIDDEN), jnp.float32),
            pltpu.SemaphoreType.DMA,
            pltpu.SemaphoreType.DMA,
            pltpu.SemaphoreType.DMA,
            pltpu.SemaphoreType.DMA,
        ],
    )
    zq = gather(idx.reshape(_NW, _NCH, _CH), emb)
    return (zq, ptr_logits)


# speedup vs baseline: 1.0705x; 1.0705x over previous
import functools

import jax
import jax.numpy as jnp
from jax import lax
from jax.experimental import pallas as pl
from jax.experimental.pallas import tpu as pltpu
from jax.experimental.pallas import tpu_sc as plsc

_HIDDEN = 896
_CODEBOOK = 2000
_CODEBOOK_PAD = 2048
_MAX_PROMPT_LEN = 128
_BATCH = 16384
_TILE = 1024

_NC = 2
_NS = 16
_NW = _NC * _NS
_B_PER_W = _BATCH // _NW
_CH = 64
_NCH = _B_PER_W // _CH


def _tc_body(z_ref, emb_ref, mb0_ref, mb1_ref, w0_ref, w1_ref,
             idx_ref, l_ref, esq_ref):
    @pl.when(pl.program_id(0) == 0)
    def _():
        emb0 = emb_ref[...]
        esq_ref[...] = jnp.sum(emb0 * emb0, axis=1)[None, :]

    z = z_ref[...]
    emb = emb_ref[...]
    mm = jax.lax.dot_general(z, emb, (((1,), (1,)), ((), ())),
                             preferred_element_type=jnp.float32)
    zz = jnp.sum(z * z, axis=1, keepdims=True)
    dist = zz - 2.0 * mm + esq_ref[...]
    dmin = jnp.min(dist, axis=1, keepdims=True)
    col = jax.lax.broadcasted_iota(jnp.int32, (_TILE, _CODEBOOK), 1)
    idx_ref[...] = jnp.min(jnp.where(dist == dmin, col, _CODEBOOK),
                           axis=1, keepdims=True)
    l_ref[:, 0, :] = jax.lax.dot_general(
        z, w0_ref[...], (((1,), (1,)), ((), ())),
        preferred_element_type=jnp.float32) + mb0_ref[...]
    l_ref[:, 1, :] = jax.lax.dot_general(
        z, w1_ref[...], (((1,), (1,)), ((), ())),
        preferred_element_type=jnp.float32) + mb1_ref[...]


def _sc_gather_body(idx_hbm, emb_hbm, out_hbm, idx_v, rows_v, gs0, gs1, ss0, ss1):
    wid = lax.axis_index("s") * _NC + lax.axis_index("c")
    base = wid * _B_PER_W
    pltpu.sync_copy(idx_hbm.at[wid], idx_v)
    gsems = (gs0, gs1)
    ssems = (ss0, ss1)
    gh = [None] * _NCH
    sh = [None] * _NCH
    gh[0] = pltpu.async_copy(emb_hbm.at[idx_v.at[0]], rows_v.at[0], gsems[0])
    for j in range(_NCH):
        b = j % 2
        if j + 1 < _NCH:
            if j - 1 >= 0:
                sh[j - 1].wait()
            gh[j + 1] = pltpu.async_copy(
                emb_hbm.at[idx_v.at[j + 1]], rows_v.at[(j + 1) % 2],
                gsems[(j + 1) % 2])
        gh[j].wait()
        sh[j] = pltpu.async_copy(
            rows_v.at[b], out_hbm.at[pl.ds(base + j * _CH, _CH)], ssems[b])
    sh[_NCH - 2].wait()
    sh[_NCH - 1].wait()


@jax.jit
def kernel(s2_premise_state, emb, W0, b0, W1, b1, prompt_len):
    z = s2_premise_state
    mask = jnp.where(jnp.arange(_MAX_PROMPT_LEN) < prompt_len,
                     jnp.float32(0.0), jnp.float32(-jnp.inf))
    mb0 = (b0 + mask)[None, :]
    mb1 = (b1 + mask)[None, :]
    grid = _BATCH // _TILE
    idx, ptr_logits = pl.pallas_call(
        _tc_body,
        grid=(grid,),
        in_specs=[
            pl.BlockSpec((_TILE, _HIDDEN), lambda i: (i, 0)),
            pl.BlockSpec((_CODEBOOK, _HIDDEN), lambda i: (0, 0)),
            pl.BlockSpec((1, _MAX_PROMPT_LEN), lambda i: (0, 0)),
            pl.BlockSpec((1, _MAX_PROMPT_LEN), lambda i: (0, 0)),
            pl.BlockSpec((_MAX_PROMPT_LEN, _HIDDEN), lambda i: (0, 0)),
            pl.BlockSpec((_MAX_PROMPT_LEN, _HIDDEN), lambda i: (0, 0)),
        ],
        out_specs=[
            pl.BlockSpec((_TILE, 1), lambda i: (i, 0)),
            pl.BlockSpec((_TILE, 2, _MAX_PROMPT_LEN), lambda i: (i, 0, 0)),
        ],
        out_shape=[
            jax.ShapeDtypeStruct((_BATCH, 1), jnp.int32),
            jax.ShapeDtypeStruct((_BATCH, 2, _MAX_PROMPT_LEN), jnp.float32),
        ],
        scratch_shapes=[pltpu.VMEM((1, _CODEBOOK), jnp.float32)],
    )(z, emb, mb0, mb1, W0, W1)

    gather = pl.kernel(
        _sc_gather_body,
        out_type=jax.ShapeDtypeStruct((_BATCH, _HIDDEN), jnp.float32),
        mesh=plsc.VectorSubcoreMesh(core_axis_name="c", subcore_axis_name="s"),
        scratch_types=[
            pltpu.VMEM((_NCH, _CH), jnp.int32),
            pltpu.VMEM((2, _CH, _HIDDEN), jnp.float32),
            pltpu.SemaphoreType.DMA,
            pltpu.SemaphoreType.DMA,
            pltpu.SemaphoreType.DMA,
            pltpu.SemaphoreType.DMA,
        ],
    )
    zq = gather(idx.reshape(_NW, _NCH, _CH), emb)
    return (zq, ptr_logits)


# speedup vs baseline: 1.0717x; 1.0011x over previous
import functools

import jax
import jax.numpy as jnp
from jax import lax
from jax.experimental import pallas as pl
from jax.experimental.pallas import tpu as pltpu
from jax.experimental.pallas import tpu_sc as plsc

_HIDDEN = 896
_CODEBOOK = 2000
_CODEBOOK_PAD = 2048
_MAX_PROMPT_LEN = 128
_BATCH = 16384
_TILE = 1024

_NC = 2
_NS = 16
_NW = _NC * _NS
_B_PER_W = _BATCH // _NW
_CH = 32
_NBUF = 4
_NCH = _B_PER_W // _CH


def _tc_body(z_ref, emb_ref, mb0_ref, mb1_ref, w0_ref, w1_ref,
             idx_ref, l_ref, esq_ref):
    @pl.when(pl.program_id(0) == 0)
    def _():
        emb0 = emb_ref[...]
        esq_ref[...] = jnp.sum(emb0 * emb0, axis=1)[None, :]

    z = z_ref[...]
    emb = emb_ref[...]
    mm = jax.lax.dot_general(z, emb, (((1,), (1,)), ((), ())),
                             preferred_element_type=jnp.float32)
    zz = jnp.sum(z * z, axis=1, keepdims=True)
    dist = zz - 2.0 * mm + esq_ref[...]
    dmin = jnp.min(dist, axis=1, keepdims=True)
    col = jax.lax.broadcasted_iota(jnp.int32, (_TILE, _CODEBOOK), 1)
    idx_ref[...] = jnp.min(jnp.where(dist == dmin, col, _CODEBOOK),
                           axis=1, keepdims=True)
    l_ref[:, 0, :] = jax.lax.dot_general(
        z, w0_ref[...], (((1,), (1,)), ((), ())),
        preferred_element_type=jnp.float32) + mb0_ref[...]
    l_ref[:, 1, :] = jax.lax.dot_general(
        z, w1_ref[...], (((1,), (1,)), ((), ())),
        preferred_element_type=jnp.float32) + mb1_ref[...]


def _sc_gather_body(idx_hbm, emb_hbm, out_hbm, idx_v, rows_v, *sems):
    wid = lax.axis_index("s") * _NC + lax.axis_index("c")
    base = wid * _B_PER_W
    pltpu.sync_copy(idx_hbm.at[wid], idx_v)
    gsems = sems[:_NBUF]
    ssems = sems[_NBUF:]
    gh = [None] * _NCH
    sh = [None] * _NCH

    def _gather(j):
        return pltpu.async_copy(emb_hbm.at[idx_v.at[j]], rows_v.at[j % _NBUF],
                                gsems[j % _NBUF])

    for j in range(min(_NBUF - 1, _NCH)):
        gh[j] = _gather(j)
    for j in range(_NCH):
        nxt = j + _NBUF - 1
        if nxt < _NCH:
            if nxt - _NBUF >= 0:
                sh[nxt - _NBUF].wait()
            gh[nxt] = _gather(nxt)
        gh[j].wait()
        sh[j] = pltpu.async_copy(
            rows_v.at[j % _NBUF], out_hbm.at[pl.ds(base + j * _CH, _CH)],
            ssems[j % _NBUF])
    for j in range(max(0, _NCH - _NBUF), _NCH):
        sh[j].wait()


@jax.jit
def kernel(s2_premise_state, emb, W0, b0, W1, b1, prompt_len):
    z = s2_premise_state
    mask = jnp.where(jnp.arange(_MAX_PROMPT_LEN) < prompt_len,
                     jnp.float32(0.0), jnp.float32(-jnp.inf))
    mb0 = (b0 + mask)[None, :]
    mb1 = (b1 + mask)[None, :]
    grid = _BATCH // _TILE
    idx, ptr_logits = pl.pallas_call(
        _tc_body,
        grid=(grid,),
        in_specs=[
            pl.BlockSpec((_TILE, _HIDDEN), lambda i: (i, 0)),
            pl.BlockSpec((_CODEBOOK, _HIDDEN), lambda i: (0, 0)),
            pl.BlockSpec((1, _MAX_PROMPT_LEN), lambda i: (0, 0)),
            pl.BlockSpec((1, _MAX_PROMPT_LEN), lambda i: (0, 0)),
            pl.BlockSpec((_MAX_PROMPT_LEN, _HIDDEN), lambda i: (0, 0)),
            pl.BlockSpec((_MAX_PROMPT_LEN, _HIDDEN), lambda i: (0, 0)),
        ],
        out_specs=[
            pl.BlockSpec((_TILE, 1), lambda i: (i, 0)),
            pl.BlockSpec((_TILE, 2, _MAX_PROMPT_LEN), lambda i: (i, 0, 0)),
        ],
        out_shape=[
            jax.ShapeDtypeStruct((_BATCH, 1), jnp.int32),
            jax.ShapeDtypeStruct((_BATCH, 2, _MAX_PROMPT_LEN), jnp.float32),
        ],
        scratch_shapes=[pltpu.VMEM((1, _CODEBOOK), jnp.float32)],
    )(z, emb, mb0, mb1, W0, W1)

    gather = pl.kernel(
        _sc_gather_body,
        out_type=jax.ShapeDtypeStruct((_BATCH, _HIDDEN), jnp.float32),
        mesh=plsc.VectorSubcoreMesh(core_axis_name="c", subcore_axis_name="s"),
        scratch_types=(
            [pltpu.VMEM((_NCH, _CH), jnp.int32),
             pltpu.VMEM((_NBUF, _CH, _HIDDEN), jnp.float32)]
            + [pltpu.SemaphoreType.DMA] * (2 * _NBUF)
        ),
    )
    zq = gather(idx.reshape(_NW, _NCH, _CH), emb)
    return (zq, ptr_logits)


# speedup vs baseline: 1.0735x; 1.0017x over previous
import jax
import jax.numpy as jnp
from jax import lax
from jax.experimental import pallas as pl
from jax.experimental.pallas import tpu as pltpu
from jax.experimental.pallas import tpu_sc as plsc

_HIDDEN = 896
_CODEBOOK = 2000
_MAX_PROMPT_LEN = 128
_BATCH = 16384
_TILE = 1024

_NC = 2
_NS = 16
_NW = _NC * _NS
_B_PER_W = _BATCH // _NW
_CH = 32
_NBUF = 4
_NCH = _B_PER_W // _CH


def _tc_body(z_ref, emb_ref, mb0_ref, mb1_ref, w0_ref, w1_ref,
             idx_ref, l_ref, esq_ref):
    @pl.when(pl.program_id(0) == 0)
    def _():
        emb0 = emb_ref[...]
        esq_ref[...] = jnp.sum(emb0 * emb0, axis=1)[None, :]

    z = z_ref[...]
    emb = emb_ref[...]
    mm = jax.lax.dot_general(z, emb, (((1,), (1,)), ((), ())),
                             preferred_element_type=jnp.float32)
    zz = jnp.sum(z * z, axis=1, keepdims=True)
    dist = zz - 2.0 * mm + esq_ref[...]
    dmin = jnp.min(dist, axis=1, keepdims=True)
    col = jax.lax.broadcasted_iota(jnp.int32, (_TILE, _CODEBOOK), 1)
    idx_ref[...] = jnp.min(jnp.where(dist == dmin, col, _CODEBOOK),
                           axis=1, keepdims=True)
    l_ref[:, 0, :] = jax.lax.dot_general(
        z, w0_ref[...], (((1,), (1,)), ((), ())),
        preferred_element_type=jnp.float32) + mb0_ref[...]
    l_ref[:, 1, :] = jax.lax.dot_general(
        z, w1_ref[...], (((1,), (1,)), ((), ())),
        preferred_element_type=jnp.float32) + mb1_ref[...]


def _sc_gather_body(idx_hbm, emb_hbm, out_hbm, idx_v, rows_v, *sems):
    wid = lax.axis_index("s") * _NC + lax.axis_index("c")
    base = wid * _B_PER_W
    pltpu.sync_copy(idx_hbm.at[wid], idx_v)
    gsems = sems[:_NBUF]
    ssems = sems[_NBUF:]
    gh = [None] * _NCH
    sh = [None] * _NCH

    def _gather(j):
        return pltpu.async_copy(emb_hbm.at[idx_v.at[j]], rows_v.at[j % _NBUF],
                                gsems[j % _NBUF])

    for j in range(min(_NBUF - 1, _NCH)):
        gh[j] = _gather(j)
    for j in range(_NCH):
        nxt = j + _NBUF - 1
        if nxt < _NCH:
            if nxt - _NBUF >= 0:
                sh[nxt - _NBUF].wait()
            gh[nxt] = _gather(nxt)
        gh[j].wait()
        sh[j] = pltpu.async_copy(
            rows_v.at[j % _NBUF], out_hbm.at[pl.ds(base + j * _CH, _CH)],
            ssems[j % _NBUF])
    for j in range(max(0, _NCH - _NBUF), _NCH):
        sh[j].wait()


@jax.jit
def kernel(s2_premise_state, emb, W0, b0, W1, b1, prompt_len):
    z = s2_premise_state
    mask = jnp.where(jnp.arange(_MAX_PROMPT_LEN) < prompt_len,
                     jnp.float32(0.0), jnp.float32(-jnp.inf))
    mb0 = (b0 + mask)[None, :]
    mb1 = (b1 + mask)[None, :]
    grid = _BATCH // _TILE
    idx, ptr_logits = pl.pallas_call(
        _tc_body,
        grid=(grid,),
        in_specs=[
            pl.BlockSpec((_TILE, _HIDDEN), lambda i: (i, 0)),
            pl.BlockSpec((_CODEBOOK, _HIDDEN), lambda i: (0, 0)),
            pl.BlockSpec((1, _MAX_PROMPT_LEN), lambda i: (0, 0)),
            pl.BlockSpec((1, _MAX_PROMPT_LEN), lambda i: (0, 0)),
            pl.BlockSpec((_MAX_PROMPT_LEN, _HIDDEN), lambda i: (0, 0)),
            pl.BlockSpec((_MAX_PROMPT_LEN, _HIDDEN), lambda i: (0, 0)),
        ],
        out_specs=[
            pl.BlockSpec((_TILE, 1), lambda i: (i, 0)),
            pl.BlockSpec((_TILE, 2, _MAX_PROMPT_LEN), lambda i: (i, 0, 0)),
        ],
        out_shape=[
            jax.ShapeDtypeStruct((_BATCH, 1), jnp.int32),
            jax.ShapeDtypeStruct((_BATCH, 2, _MAX_PROMPT_LEN), jnp.float32),
        ],
        scratch_shapes=[pltpu.VMEM((1, _CODEBOOK), jnp.float32)],
    )(z, emb, mb0, mb1, W0, W1)

    gather = pl.kernel(
        _sc_gather_body,
        out_type=jax.ShapeDtypeStruct((_BATCH, _HIDDEN), jnp.float32),
        mesh=plsc.VectorSubcoreMesh(core_axis_name="c", subcore_axis_name="s"),
        scratch_types=(
            [pltpu.VMEM((_NCH, _CH), jnp.int32),
             pltpu.VMEM((_NBUF, _CH, _HIDDEN), jnp.float32)]
            + [pltpu.SemaphoreType.DMA] * (2 * _NBUF)
        ),
    )
    zq = gather(idx.reshape(_NW, _NCH, _CH), emb)
    return (zq, ptr_logits)
